# Initial kernel scaffold; baseline (speedup 1.0000x reference)
import jax
import jax.numpy as jnp
from jax.experimental import pallas as pl


def kernel(token_id, table):
    raise NotImplementedError("write your pallas kernel here")


# speedup vs baseline: 1.8427x; 1.8427x over previous
import functools

import jax
import jax.numpy as jnp
from jax import lax
from jax.experimental import pallas as pl
from jax.experimental.pallas import tpu as pltpu
from jax.experimental.pallas import tpu_sc as plsc

NUM_WORKERS = 32
CHUNK = 1024


def _emb_body(idx_hbm, table_hbm, out_hbm, idx_v, rows_v, sem, *, per_w):
    wid = lax.axis_index("s") * 2 + lax.axis_index("c")
    base = wid * per_w

    def step(i, carry):
        off = base + i * CHUNK
        pltpu.sync_copy(idx_hbm.at[pl.ds(off, CHUNK)], idx_v)
        pltpu.async_copy(table_hbm.at[idx_v], rows_v, sem).wait()
        pltpu.sync_copy(rows_v, out_hbm.at[pl.ds(off, CHUNK)])
        return carry

    lax.fori_loop(0, per_w // CHUNK, step, 0)


def kernel(token_id, table):
    B, H = token_id.shape
    V, D = table.shape
    N = B * H
    per_w = N // NUM_WORKERS
    idx = token_id.reshape(N).astype(jnp.int32)

    mesh = plsc.VectorSubcoreMesh(core_axis_name="c", subcore_axis_name="s")
    emb = functools.partial(
        pl.kernel,
        mesh=mesh,
        out_type=jax.ShapeDtypeStruct((N, D), jnp.float32),
        scratch_types=[
            pltpu.VMEM((CHUNK,), jnp.int32),
            pltpu.VMEM((CHUNK, D), jnp.float32),
            pltpu.SemaphoreType.DMA,
        ],
        compiler_params=pltpu.CompilerParams(use_tc_tiling_on_sc=False),
    )(functools.partial(_emb_body, per_w=per_w))

    out = emb(idx, table)
    return out.reshape(B, H, D)


# speedup vs baseline: 1.8756x; 1.0179x over previous
import functools

import jax
import jax.numpy as jnp
from jax import lax
from jax.experimental import pallas as pl
from jax.experimental.pallas import tpu as pltpu
from jax.experimental.pallas import tpu_sc as plsc

NUM_WORKERS = 32
CHUNK = 800


def _emb_body(idx_hbm, table_hbm, out_hbm, idx_v, rows_v, g0, g1, o0, o1,
              *, per_w, nchunks):
    wid = lax.axis_index("s") * 2 + lax.axis_index("c")
    base = wid * per_w
    pltpu.sync_copy(idx_hbm.at[pl.ds(base, per_w)], idx_v)
    gsem = (g0, g1)
    osem = (o0, o1)

    def g_start(g, b):
        pltpu.make_async_copy(table_hbm.at[idx_v.at[pl.ds(g * CHUNK, CHUNK)]],
                              rows_v.at[b], gsem[b]).start()

    def g_wait(b):
        pltpu.make_async_copy(table_hbm.at[idx_v.at[pl.ds(0, CHUNK)]],
                              rows_v.at[b], gsem[b]).wait()

    def w_start(g, b):
        pltpu.make_async_copy(rows_v.at[b],
                              out_hbm.at[pl.ds(base + g * CHUNK, CHUNK)],
                              osem[b]).start()

    def w_wait(b):
        pltpu.make_async_copy(rows_v.at[b],
                              out_hbm.at[pl.ds(base, CHUNK)], osem[b]).wait()

    g_start(0, 0)
    g_start(1, 1)
    g_wait(0)
    w_start(0, 0)

    def pair(go, carry):
        u = 2 * go + 1
        g_wait(1)
        w_start(u, 1)
        w_wait(0)
        g_start(u + 1, 0)
        g_wait(0)
        w_start(u + 1, 0)
        w_wait(1)
        g_start(u + 2, 1)
        return carry

    lax.fori_loop(0, (nchunks - 2) // 2, pair, 0)

    g_wait(1)
    w_start(nchunks - 1, 1)
    w_wait(0)
    w_wait(1)


def kernel(token_id, table):
    B, H = token_id.shape
    V, D = table.shape
    N = B * H
    per_w = N // NUM_WORKERS
    nchunks = per_w // CHUNK
    idx = token_id.reshape(N).astype(jnp.int32)

    mesh = plsc.VectorSubcoreMesh(core_axis_name="c", subcore_axis_name="s")
    emb = functools.partial(
        pl.kernel,
        mesh=mesh,
        out_type=jax.ShapeDtypeStruct((N, D), jnp.float32),
        scratch_types=[
            pltpu.VMEM((per_w,), jnp.int32),
            pltpu.VMEM((2, CHUNK, D), jnp.float32),
            pltpu.SemaphoreType.DMA,
            pltpu.SemaphoreType.DMA,
            pltpu.SemaphoreType.DMA,
            pltpu.SemaphoreType.DMA,
        ],
        compiler_params=pltpu.CompilerParams(use_tc_tiling_on_sc=False),
    )(functools.partial(_emb_body, per_w=per_w, nchunks=nchunks))

    out = emb(idx, table)
    return out.reshape(B, H, D)
